# Initial kernel scaffold; baseline (speedup 1.0000x reference)
import jax
import jax.numpy as jnp
from jax.experimental import pallas as pl


def kernel(x, edge_index, batch, W_enc, b_enc, W_root, W_rel, b_score, gamma, beta, W_fc, b_fc):
    raise NotImplementedError("write your pallas kernel here")


# speedup vs baseline: 51.7561x; 51.7561x over previous
import functools

import jax
import jax.numpy as jnp
from jax import lax
from jax.experimental import pallas as pl
from jax.experimental.pallas import tpu as pltpu
from jax.experimental.pallas import tpu_sc as plsc

N = 10000
E = 320000
DIN = 128
H = 512
C = 4
G = 64

NW = 32
CH = 79
CW = 128
EPAD = NW * CH * CW
STRIPE = 640
NSC = 16 * STRIPE
NPR = 80
NP = NPR * 128
BR = 1024
NB = NP // BR

_mesh = plsc.VectorSubcoreMesh(core_axis_name="c", subcore_axis_name="s")
_f32 = jnp.float32


@functools.partial(
    pl.kernel,
    out_type=jax.ShapeDtypeStruct((2 * NSC,), _f32),
    mesh=_mesh,
    scratch_types=[
        pltpu.VMEM((CH, CW), jnp.int32),
        pltpu.VMEM((CW,), _f32),
        pltpu.VMEM((STRIPE,), _f32),
        pltpu.VMEM_SHARED((NSC,), _f32),
    ],
)
def _sc_deg(dst_hbm, out_hbm, dst_v, ones_v, zbuf, acc_sh):
    c = lax.axis_index("c")
    s = lax.axis_index("s")
    wid = s * 2 + c
    pltpu.sync_copy(dst_hbm.at[wid], dst_v)
    for j in range(CW // 16):
        ones_v[pl.ds(j * 16, 16)] = jnp.ones((16,), _f32)

    def zfill(i, carry):
        zbuf[pl.ds(i * 16, 16)] = jnp.zeros((16,), _f32)
        return carry

    lax.fori_loop(0, STRIPE // 16, zfill, 0)
    pltpu.sync_copy(zbuf, acc_sh.at[pl.ds(s * STRIPE, STRIPE)])
    plsc.subcore_barrier()

    def body(i, carry):
        pltpu.sync_copy(ones_v, acc_sh.at[dst_v.at[i]], add=True)
        return carry

    lax.fori_loop(0, CH, body, 0)
    plsc.subcore_barrier()
    pltpu.sync_copy(acc_sh.at[pl.ds(s * STRIPE, STRIPE)],
                    out_hbm.at[pl.ds(c * NSC + s * STRIPE, STRIPE)])


@functools.partial(
    pl.kernel,
    out_type=jax.ShapeDtypeStruct((2 * NSC, DIN), _f32),
    mesh=_mesh,
    scratch_types=[
        pltpu.VMEM((CW,), jnp.int32),
        pltpu.VMEM((CW,), jnp.int32),
        pltpu.VMEM((CH, CW), jnp.int32),
        pltpu.VMEM((CW, DIN), _f32),
        pltpu.VMEM((CW, DIN), _f32),
        pltpu.VMEM_SHARED((NSC, DIN), _f32),
        pltpu.SemaphoreType.DMA,
        pltpu.SemaphoreType.DMA,
    ],
)
def _sc_agg(src_hbm, dst_hbm, xs_hbm, out_hbm,
            src_a, src_b, dst_v, rows_a, rows_b, acc_sh, sem_a, sem_b):
    c = lax.axis_index("c")
    s = lax.axis_index("s")
    wid = s * 2 + c
    pltpu.sync_copy(dst_hbm.at[wid], dst_v)

    def zfill(r, carry):
        for j in range(DIN // 16):
            rows_a[r, pl.ds(j * 16, 16)] = jnp.zeros((16,), _f32)
        return carry

    lax.fori_loop(0, CW, zfill, 0)
    for i in range(STRIPE // CW):
        pltpu.sync_copy(rows_a, acc_sh.at[pl.ds(s * STRIPE + i * CW, CW), :])
    plsc.subcore_barrier()

    pltpu.sync_copy(src_hbm.at[wid, 0], src_a)
    pltpu.async_copy(xs_hbm.at[src_a], rows_a, sem_a)

    def pair(t, carry):
        i = 2 * t

        @pl.when(i + 1 < CH)
        def _():
            pltpu.sync_copy(src_hbm.at[wid, i + 1], src_b)
            pltpu.async_copy(xs_hbm.at[src_b], rows_b, sem_b)

        pltpu.make_async_copy(xs_hbm.at[src_a], rows_a, sem_a).wait()
        pltpu.sync_copy(rows_a, acc_sh.at[dst_v.at[i]], add=True)

        @pl.when(i + 2 < CH)
        def _():
            pltpu.sync_copy(src_hbm.at[wid, i + 2], src_a)
            pltpu.async_copy(xs_hbm.at[src_a], rows_a, sem_a)

        @pl.when(i + 1 < CH)
        def _():
            pltpu.make_async_copy(xs_hbm.at[src_b], rows_b, sem_b).wait()
            pltpu.sync_copy(rows_b, acc_sh.at[dst_v.at[i + 1]], add=True)

        return carry

    lax.fori_loop(0, (CH + 1) // 2, pair, 0)
    plsc.subcore_barrier()
    pltpu.sync_copy(acc_sh.at[pl.ds(s * STRIPE, STRIPE), :],
                    out_hbm.at[pl.ds(c * NSC + s * STRIPE, STRIPE), :])


@functools.partial(
    pl.kernel,
    out_type=jax.ShapeDtypeStruct((2 * NSC,), _f32),
    mesh=_mesh,
    scratch_types=[
        pltpu.VMEM((CH, CW), jnp.int32),
        pltpu.VMEM((CH, CW), jnp.int32),
        pltpu.VMEM((STRIPE,), _f32),
        pltpu.VMEM((CW,), _f32),
        pltpu.VMEM_SHARED((NSC,), _f32),
        pltpu.VMEM_SHARED((NSC,), _f32),
        pltpu.SemaphoreType.DMA,
    ],
)
def _sc_sagg(src_hbm, dst_hbm, s2_hbm, out_hbm,
             src_v, dst_v, sbuf, gbuf, s2_sh, acc_sh, sem):
    c = lax.axis_index("c")
    s = lax.axis_index("s")
    wid = s * 2 + c
    pltpu.sync_copy(src_hbm.at[wid], src_v)
    pltpu.sync_copy(dst_hbm.at[wid], dst_v)
    pltpu.sync_copy(s2_hbm.at[pl.ds(s * STRIPE, STRIPE)], sbuf)
    pltpu.sync_copy(sbuf, s2_sh.at[pl.ds(s * STRIPE, STRIPE)])

    def zfill(i, carry):
        sbuf[pl.ds(i * 16, 16)] = jnp.zeros((16,), _f32)
        return carry

    lax.fori_loop(0, STRIPE // 16, zfill, 0)
    pltpu.sync_copy(sbuf, acc_sh.at[pl.ds(s * STRIPE, STRIPE)])
    plsc.subcore_barrier()

    def body(i, carry):
        pltpu.async_copy(s2_sh.at[src_v.at[i]], gbuf, sem).wait()
        pltpu.sync_copy(gbuf, acc_sh.at[dst_v.at[i]], add=True)
        return carry

    lax.fori_loop(0, CH, body, 0)
    plsc.subcore_barrier()
    pltpu.sync_copy(acc_sh.at[pl.ds(s * STRIPE, STRIPE)],
                    out_hbm.at[pl.ds(c * NSC + s * STRIPE, STRIPE)])


def _tc_scale_body(x_ref, d0_ref, d1_ref, xs_ref, dinv_ref):
    deg = d0_ref[...] + d1_ref[...] + 1.0
    dinv = lax.rsqrt(deg)
    dinv_ref[...] = dinv
    dcol = jnp.transpose(dinv.reshape(1, NP), (1, 0))
    xs_ref[...] = x_ref[...] * dcol


def _tc_scale(xp, d0, d1):
    return pl.pallas_call(
        _tc_scale_body,
        out_shape=[
            jax.ShapeDtypeStruct((NP, DIN), _f32),
            jax.ShapeDtypeStruct((NPR, 128), _f32),
        ],
    )(xp, d0, d1)


def _tc_encode_body(a0_ref, a1_ref, x_ref, dv_ref, We_ref, be_ref, Wrr_ref,
                    h_ref, s12_ref):
    dcol = jnp.transpose(dv_ref[...].reshape(1, BR), (1, 0))
    h_in = (a0_ref[...] + a1_ref[...]) * dcol + x_ref[...] * (dcol * dcol)
    hb = jnp.dot(h_in, We_ref[...], preferred_element_type=_f32)
    hb = jnp.maximum(hb + be_ref[...][0:1, :], 0.0)
    h_ref[...] = hb
    s12_ref[...] = jnp.dot(hb, Wrr_ref[...], preferred_element_type=_f32)


def _tc_encode(a0, a1, xp, dinv2d, W_enc, be2d, Wrr):
    blk = lambda i: (i, 0)
    zero = lambda i: (0, 0)
    return pl.pallas_call(
        _tc_encode_body,
        grid=(NB,),
        in_specs=[
            pl.BlockSpec((BR, DIN), blk),
            pl.BlockSpec((BR, DIN), blk),
            pl.BlockSpec((BR, DIN), blk),
            pl.BlockSpec((8, 128), blk),
            pl.BlockSpec((DIN, H), zero),
            pl.BlockSpec((8, H), zero),
            pl.BlockSpec((H, 128), zero),
        ],
        out_specs=[
            pl.BlockSpec((BR, H), blk),
            pl.BlockSpec((BR, 128), blk),
        ],
        out_shape=[
            jax.ShapeDtypeStruct((NP, H), _f32),
            jax.ShapeDtypeStruct((NP, 128), _f32),
        ],
    )(a0, a1, xp, dinv2d, W_enc, be2d, Wrr)


def _tc_final_body(h_ref, s1_ref, sg0_ref, sg1_ref, batch_ref, bs_ref,
                   gamma_ref, beta_ref, Wfc_ref, bfc_ref, out_ref,
                   w_scr, cnt_scr, S_scr, Q_scr):
    pid = pl.program_id(0)

    @pl.when(pid == 0)
    def _phase1():
        score2d = s1_ref[...] + sg0_ref[...] + sg1_ref[...] + bs_ref[...]
        score2d = jnp.where(score2d == 0.0, 0.0, score2d)
        score_row = score2d.reshape(1, NP)
        batch_row = batch_ref[...].reshape(1, NP)
        g_col = lax.broadcasted_iota(jnp.int32, (G, 1), 0)
        onehot = batch_row == g_col
        cnt = jnp.sum(jnp.where(onehot, 1.0, 0.0), axis=1, keepdims=True)
        kq = jnp.floor((cnt + 1.0) * 0.5)

        sb = lax.bitcast_convert_type(score_row, jnp.int32)
        key_row = sb ^ (lax.shift_right_arithmetic(sb, 31)
                        & jnp.int32(0x7FFFFFFF))

        def count_ge(tcol):
            ge = (key_row >= tcol) & onehot
            return jnp.sum(jnp.where(ge, 1.0, 0.0), axis=1, keepdims=True)

        zero_t = jnp.zeros((G, 1), jnp.int32)
        t0 = jnp.where(count_ge(zero_t) >= kq, zero_t,
                       jnp.full((G, 1), jnp.int32(-2147483648)))

        def bit_body(bi, t):
            tc = t + (jnp.int32(1) << (30 - bi))
            return jnp.where(count_ge(tc) >= kq, tc, t)

        tfin = lax.fori_loop(0, 31, bit_body, t0)

        gt_m = (key_row > tfin) & onehot
        eq_m = (key_row == tfin) & onehot
        cnt_gt = jnp.sum(jnp.where(gt_m, 1.0, 0.0), axis=1, keepdims=True)
        eqc = jnp.sum(jnp.where(eq_m, 1.0, 0.0), axis=1, keepdims=True)
        need = kq - cnt_gt

        gi = lax.broadcasted_iota(jnp.int32, (G, G), 0)
        gj = lax.broadcasted_iota(jnp.int32, (G, G), 1)
        tri_g = jnp.where(gj < gi, 1.0, 0.0)
        base = jnp.dot(tri_g, eqc, preferred_element_type=_f32)
        lim = base + need

        eq_row = jnp.sum(jnp.where(eq_m, 1.0, 0.0), axis=0, keepdims=True)
        eq2d = eq_row.reshape(NPR, 128)
        ci = lax.broadcasted_iota(jnp.int32, (128, 128), 0)
        cj = lax.broadcasted_iota(jnp.int32, (128, 128), 1)
        tri_incl = jnp.where(ci <= cj, 1.0, 0.0)
        cum_in_row = jnp.dot(eq2d, tri_incl, preferred_element_type=_f32)
        rowtot = jnp.sum(eq2d, axis=1, keepdims=True)
        ri = lax.broadcasted_iota(jnp.int32, (NPR, NPR), 0)
        rj = lax.broadcasted_iota(jnp.int32, (NPR, NPR), 1)
        tri_r = jnp.where(rj < ri, 1.0, 0.0)
        rowbase = jnp.dot(tri_r, rowtot, preferred_element_type=_f32)
        p_excl = cum_in_row + rowbase - eq2d
        p_row = p_excl.reshape(1, NP)

        sel_m = gt_m | (eq_m & (p_row < lim))
        sel_row = jnp.sum(jnp.where(sel_m, 1.0, 0.0), axis=0, keepdims=True)
        w_row = jnp.tanh(score_row) * sel_row
        w_scr[...] = w_row.reshape(NPR, 128)
        cnt_scr[...] = jnp.broadcast_to(kq, (G, 128))
        S_scr[...] = jnp.zeros((G, H), _f32)
        Q_scr[...] = jnp.zeros((8, H), _f32)

    hb = h_ref[...]
    wcol = jnp.transpose(w_scr[pl.ds(pid * 8, 8), :].reshape(1, BR), (1, 0))
    brow = batch_ref[pl.ds(pid * 8, 8), :].reshape(1, BR)
    g_col = lax.broadcasted_iota(jnp.int32, (G, 1), 0)
    onehot_t = jnp.where(brow == g_col, 1.0, 0.0)
    hw = hb * wcol
    S_scr[...] += jnp.dot(onehot_t, hw, preferred_element_type=_f32)
    Q_scr[0:1, :] += jnp.sum(hw * hw, axis=0, keepdims=True)

    @pl.when(pid == NB - 1)
    def _finalize():
        S = S_scr[...]
        cnt = cnt_scr[:, 0:1]
        m_tot = jnp.sum(cnt)
        mean = jnp.sum(S, axis=0, keepdims=True) / m_tot
        var = Q_scr[0:1, :] / m_tot - mean * mean
        gm = gamma_ref[...][0:1, :] * lax.rsqrt(var + 1e-5)
        cntc = jnp.maximum(cnt, 1.0)
        pooled = (S / cntc - mean) * gm + beta_ref[...][0:1, :]
        pooled = jnp.where(cnt > 0.0, pooled, 0.0)
        logits = jnp.dot(pooled, Wfc_ref[...], preferred_element_type=_f32)
        logits = logits + bfc_ref[...][0:1, :]
        lmask = lax.broadcasted_iota(jnp.int32, (1, 128), 1) < C
        lg = jnp.where(lmask, logits, jnp.float32(-1e30))
        mx = jnp.max(lg, axis=1, keepdims=True)
        z = lg - mx
        ez = jnp.where(lmask, jnp.exp(z), 0.0)
        lse = jnp.log(jnp.sum(ez, axis=1, keepdims=True))
        out_ref[...] = z - lse


def _tc_final(hp, s1_2d, sg0, sg1, batchp, bs2d, gm2d, bt2d, Wfc, bfc2d):
    blk = lambda i: (i, 0)
    zero = lambda i: (0, 0)
    return pl.pallas_call(
        _tc_final_body,
        grid=(NB,),
        in_specs=[
            pl.BlockSpec((BR, H), blk),
            pl.BlockSpec((NPR, 128), zero),
            pl.BlockSpec((NPR, 128), zero),
            pl.BlockSpec((NPR, 128), zero),
            pl.BlockSpec((NPR, 128), zero),
            pl.BlockSpec((NPR, 128), zero),
            pl.BlockSpec((8, H), zero),
            pl.BlockSpec((8, H), zero),
            pl.BlockSpec((H, 128), zero),
            pl.BlockSpec((8, 128), zero),
        ],
        out_specs=pl.BlockSpec((G, 128), zero),
        out_shape=jax.ShapeDtypeStruct((G, 128), _f32),
        scratch_shapes=[
            pltpu.VMEM((NPR, 128), _f32),
            pltpu.VMEM((G, 128), _f32),
            pltpu.VMEM((G, H), _f32),
            pltpu.VMEM((8, H), _f32),
        ],
    )(hp, s1_2d, sg0, sg1, batchp, bs2d, gm2d, bt2d, Wfc, bfc2d)


def _to2d(v):
    return v.reshape(NPR, 128)


def kernel(x, edge_index, batch, W_enc, b_enc, W_root, W_rel, b_score,
           gamma, beta, W_fc, b_fc):
    src = edge_index[0]
    dst = edge_index[1]
    npad = EPAD - E
    padidx = (jnp.arange(npad, dtype=jnp.int32) % 48) + N
    srcp = jnp.concatenate([src, padidx]).reshape(NW, CH, CW)
    dstp = jnp.concatenate([dst, padidx]).reshape(NW, CH, CW)

    xp = jnp.pad(x, ((0, NP - N), (0, 0)))
    batchp = jnp.pad(batch, (0, NP - N), constant_values=G).reshape(NPR, 128)

    deg_parts = _sc_deg(dstp)
    d0 = _to2d(deg_parts[:NSC])
    d1 = _to2d(deg_parts[NSC:])

    xs, dinv2d = _tc_scale(xp, d0, d1)

    agg_parts = _sc_agg(srcp, dstp, xs)
    a0 = agg_parts[:NSC]
    a1 = agg_parts[NSC:]

    be2d = jnp.broadcast_to(b_enc.reshape(1, H), (8, H))
    Wrr = jnp.pad(jnp.concatenate([W_root, W_rel], axis=1), ((0, 0), (0, 126)))
    hp, s12 = _tc_encode(a0, a1, xp, dinv2d, W_enc, be2d, Wrr)

    s2p = jnp.pad(s12[:N, 1], (0, NSC - N))
    sagg_parts = _sc_sagg(srcp, dstp, s2p)
    sg0 = _to2d(sagg_parts[:NSC])
    sg1 = _to2d(sagg_parts[NSC:])

    s1_2d = jnp.pad(s12[:N, 0], (0, NP - N)).reshape(NPR, 128)
    bs2d = jnp.broadcast_to(b_score.reshape(1, 1), (NPR, 128))
    gm2d = jnp.broadcast_to(gamma.reshape(1, H), (8, H))
    bt2d = jnp.broadcast_to(beta.reshape(1, H), (8, H))
    Wfc = jnp.pad(W_fc, ((0, 0), (0, 128 - C)))
    bfc2d = jnp.broadcast_to(jnp.pad(b_fc, (0, 128 - C)).reshape(1, 128),
                             (8, 128))

    out2d = _tc_final(hp, s1_2d, sg0, sg1, batchp, bs2d, gm2d, bt2d,
                      Wfc, bfc2d)
    return out2d[:, :C]


# speedup vs baseline: 53.9047x; 1.0415x over previous
import functools

import jax
import jax.numpy as jnp
from jax import lax
from jax.experimental import pallas as pl
from jax.experimental.pallas import tpu as pltpu
from jax.experimental.pallas import tpu_sc as plsc

N = 10000
E = 320000
DIN = 128
H = 512
C = 4
G = 64

NW = 32
CH = 79
CW = 128
EPAD = NW * CH * CW
STRIPE = 640
NSC = 16 * STRIPE
NPR = 80
NP = NPR * 128
BR = 1024
NB = NP // BR

_mesh = plsc.VectorSubcoreMesh(core_axis_name="c", subcore_axis_name="s")
_f32 = jnp.float32


@functools.partial(
    pl.kernel,
    out_type=jax.ShapeDtypeStruct((2 * NSC,), _f32),
    mesh=_mesh,
    scratch_types=[
        pltpu.VMEM((CH, CW), jnp.int32),
        pltpu.VMEM((CW,), _f32),
        pltpu.VMEM((STRIPE,), _f32),
        pltpu.VMEM_SHARED((NSC,), _f32),
    ],
)
def _sc_deg(dst_hbm, out_hbm, dst_v, ones_v, zbuf, acc_sh):
    c = lax.axis_index("c")
    s = lax.axis_index("s")
    wid = s * 2 + c
    pltpu.sync_copy(dst_hbm.at[wid], dst_v)
    for j in range(CW // 16):
        ones_v[pl.ds(j * 16, 16)] = jnp.ones((16,), _f32)

    def zfill(i, carry):
        zbuf[pl.ds(i * 16, 16)] = jnp.zeros((16,), _f32)
        return carry

    lax.fori_loop(0, STRIPE // 16, zfill, 0)
    pltpu.sync_copy(zbuf, acc_sh.at[pl.ds(s * STRIPE, STRIPE)])
    plsc.subcore_barrier()

    def body(i, carry):
        pltpu.sync_copy(ones_v, acc_sh.at[dst_v.at[i]], add=True)
        return carry

    lax.fori_loop(0, CH, body, 0)
    plsc.subcore_barrier()
    pltpu.sync_copy(acc_sh.at[pl.ds(s * STRIPE, STRIPE)],
                    out_hbm.at[pl.ds(c * NSC + s * STRIPE, STRIPE)])


@functools.partial(
    pl.kernel,
    out_type=jax.ShapeDtypeStruct((2 * NSC, DIN), _f32),
    mesh=_mesh,
    scratch_types=[
        pltpu.VMEM((CW,), jnp.int32),
        pltpu.VMEM((CW,), jnp.int32),
        pltpu.VMEM((CH, CW), jnp.int32),
        pltpu.VMEM((CW, DIN), _f32),
        pltpu.VMEM((CW, DIN), _f32),
        pltpu.VMEM_SHARED((NSC, DIN), _f32),
        pltpu.SemaphoreType.DMA,
        pltpu.SemaphoreType.DMA,
    ],
)
def _sc_agg(src_hbm, dst_hbm, xs_hbm, out_hbm,
            src_a, src_b, dst_v, rows_a, rows_b, acc_sh, sem_a, sem_b):
    c = lax.axis_index("c")
    s = lax.axis_index("s")
    wid = s * 2 + c
    pltpu.sync_copy(dst_hbm.at[wid], dst_v)

    def zfill(r, carry):
        for j in range(DIN // 16):
            rows_a[r, pl.ds(j * 16, 16)] = jnp.zeros((16,), _f32)
        return carry

    lax.fori_loop(0, CW, zfill, 0)
    for i in range(STRIPE // CW):
        pltpu.sync_copy(rows_a, acc_sh.at[pl.ds(s * STRIPE + i * CW, CW), :])
    plsc.subcore_barrier()

    pltpu.sync_copy(src_hbm.at[wid, 0], src_a)
    pltpu.async_copy(xs_hbm.at[src_a], rows_a, sem_a)

    def pair(t, carry):
        i = 2 * t

        @pl.when(i + 1 < CH)
        def _():
            pltpu.sync_copy(src_hbm.at[wid, i + 1], src_b)
            pltpu.async_copy(xs_hbm.at[src_b], rows_b, sem_b)

        pltpu.make_async_copy(xs_hbm.at[src_a], rows_a, sem_a).wait()
        pltpu.sync_copy(rows_a, acc_sh.at[dst_v.at[i]], add=True)

        @pl.when(i + 2 < CH)
        def _():
            pltpu.sync_copy(src_hbm.at[wid, i + 2], src_a)
            pltpu.async_copy(xs_hbm.at[src_a], rows_a, sem_a)

        @pl.when(i + 1 < CH)
        def _():
            pltpu.make_async_copy(xs_hbm.at[src_b], rows_b, sem_b).wait()
            pltpu.sync_copy(rows_b, acc_sh.at[dst_v.at[i + 1]], add=True)

        return carry

    lax.fori_loop(0, (CH + 1) // 2, pair, 0)
    plsc.subcore_barrier()
    pltpu.sync_copy(acc_sh.at[pl.ds(s * STRIPE, STRIPE), :],
                    out_hbm.at[pl.ds(c * NSC + s * STRIPE, STRIPE), :])


@functools.partial(
    pl.kernel,
    out_type=jax.ShapeDtypeStruct((2 * NSC,), _f32),
    mesh=_mesh,
    scratch_types=[
        pltpu.VMEM((CH, CW), jnp.int32),
        pltpu.VMEM((CH, CW), jnp.int32),
        pltpu.VMEM((STRIPE,), _f32),
        pltpu.VMEM((CW,), _f32),
        pltpu.VMEM((CW,), _f32),
        pltpu.VMEM_SHARED((NSC,), _f32),
        pltpu.VMEM_SHARED((NSC,), _f32),
        pltpu.SemaphoreType.DMA,
        pltpu.SemaphoreType.DMA,
    ],
)
def _sc_sagg(src_hbm, dst_hbm, s2_hbm, out_hbm,
             src_v, dst_v, sbuf, gbuf_a, gbuf_b, s2_sh, acc_sh, sem_a, sem_b):
    c = lax.axis_index("c")
    s = lax.axis_index("s")
    wid = s * 2 + c
    pltpu.sync_copy(src_hbm.at[wid], src_v)
    pltpu.sync_copy(dst_hbm.at[wid], dst_v)
    pltpu.sync_copy(s2_hbm.at[pl.ds(s * STRIPE, STRIPE)], sbuf)
    pltpu.sync_copy(sbuf, s2_sh.at[pl.ds(s * STRIPE, STRIPE)])

    def zfill(i, carry):
        sbuf[pl.ds(i * 16, 16)] = jnp.zeros((16,), _f32)
        return carry

    lax.fori_loop(0, STRIPE // 16, zfill, 0)
    pltpu.sync_copy(sbuf, acc_sh.at[pl.ds(s * STRIPE, STRIPE)])
    plsc.subcore_barrier()

    pltpu.async_copy(s2_sh.at[src_v.at[0]], gbuf_a, sem_a)

    def pair(t, carry):
        i = 2 * t

        @pl.when(i + 1 < CH)
        def _():
            pltpu.async_copy(s2_sh.at[src_v.at[i + 1]], gbuf_b, sem_b)

        pltpu.make_async_copy(s2_sh.at[src_v.at[i]], gbuf_a, sem_a).wait()
        pltpu.sync_copy(gbuf_a, acc_sh.at[dst_v.at[i]], add=True)

        @pl.when(i + 2 < CH)
        def _():
            pltpu.async_copy(s2_sh.at[src_v.at[i + 2]], gbuf_a, sem_a)

        @pl.when(i + 1 < CH)
        def _():
            pltpu.make_async_copy(s2_sh.at[src_v.at[i + 1]], gbuf_b, sem_b).wait()
            pltpu.sync_copy(gbuf_b, acc_sh.at[dst_v.at[i + 1]], add=True)

        return carry

    lax.fori_loop(0, (CH + 1) // 2, pair, 0)
    plsc.subcore_barrier()
    pltpu.sync_copy(acc_sh.at[pl.ds(s * STRIPE, STRIPE)],
                    out_hbm.at[pl.ds(c * NSC + s * STRIPE, STRIPE)])


def _tc_scale_body(x_ref, d0_ref, d1_ref, xs_ref, dinv_ref):
    deg = d0_ref[...] + d1_ref[...] + 1.0
    dinv = lax.rsqrt(deg)
    dinv_ref[...] = dinv
    dcol = jnp.transpose(dinv.reshape(1, NP), (1, 0))
    xs_ref[...] = x_ref[...] * dcol


def _tc_scale(xp, d0, d1):
    return pl.pallas_call(
        _tc_scale_body,
        out_shape=[
            jax.ShapeDtypeStruct((NP, DIN), _f32),
            jax.ShapeDtypeStruct((NPR, 128), _f32),
        ],
    )(xp, d0, d1)


def _tc_encode_body(a0_ref, a1_ref, x_ref, dv_ref, We_ref, be_ref, Wrr_ref,
                    h_ref, s12_ref):
    dcol = jnp.transpose(dv_ref[...].reshape(1, BR), (1, 0))
    h_in = (a0_ref[...] + a1_ref[...]) * dcol + x_ref[...] * (dcol * dcol)
    hb = jnp.dot(h_in, We_ref[...], preferred_element_type=_f32)
    hb = jnp.maximum(hb + be_ref[...][0:1, :], 0.0)
    h_ref[...] = hb
    s12_ref[...] = jnp.dot(hb, Wrr_ref[...], preferred_element_type=_f32)


def _tc_encode(a0, a1, xp, dinv2d, W_enc, be2d, Wrr):
    blk = lambda i: (i, 0)
    zero = lambda i: (0, 0)
    return pl.pallas_call(
        _tc_encode_body,
        grid=(NB,),
        in_specs=[
            pl.BlockSpec((BR, DIN), blk),
            pl.BlockSpec((BR, DIN), blk),
            pl.BlockSpec((BR, DIN), blk),
            pl.BlockSpec((8, 128), blk),
            pl.BlockSpec((DIN, H), zero),
            pl.BlockSpec((8, H), zero),
            pl.BlockSpec((H, 128), zero),
        ],
        out_specs=[
            pl.BlockSpec((BR, H), blk),
            pl.BlockSpec((BR, 128), blk),
        ],
        out_shape=[
            jax.ShapeDtypeStruct((NP, H), _f32),
            jax.ShapeDtypeStruct((NP, 128), _f32),
        ],
    )(a0, a1, xp, dinv2d, W_enc, be2d, Wrr)


def _tc_final_body(h_ref, s1_ref, sg0_ref, sg1_ref, batch_ref, bs_ref,
                   gamma_ref, beta_ref, Wfc_ref, bfc_ref, out_ref,
                   w_scr, cnt_scr, S_scr, Q_scr):
    pid = pl.program_id(0)

    @pl.when(pid == 0)
    def _phase1():
        score2d = s1_ref[...] + sg0_ref[...] + sg1_ref[...] + bs_ref[...]
        score2d = jnp.where(score2d == 0.0, 0.0, score2d)
        score_row = score2d.reshape(1, NP)
        batch_row = batch_ref[...].reshape(1, NP)
        g_col = lax.broadcasted_iota(jnp.int32, (G, 1), 0)
        onehot = batch_row == g_col
        cnt = jnp.sum(jnp.where(onehot, 1.0, 0.0), axis=1, keepdims=True)
        kq = jnp.floor((cnt + 1.0) * 0.5)

        sb = lax.bitcast_convert_type(score_row, jnp.int32)
        key_row = sb ^ (lax.shift_right_arithmetic(sb, 31)
                        & jnp.int32(0x7FFFFFFF))

        key_m = jnp.where(onehot, key_row, jnp.int32(-2147483648))

        def count_ge(tcol):
            return jnp.sum(jnp.where(key_m >= tcol, 1.0, 0.0), axis=1,
                           keepdims=True)

        zero_t = jnp.zeros((G, 1), jnp.int32)
        t0 = jnp.where(count_ge(zero_t) >= kq, zero_t,
                       jnp.full((G, 1), jnp.int32(-2147483648)))

        def bit_body(bi, t):
            tc = t + (jnp.int32(1) << (30 - bi))
            return jnp.where(count_ge(tc) >= kq, tc, t)

        tfin = lax.fori_loop(0, 31, bit_body, t0)

        gt_m = key_m > tfin
        eq_m = (key_m == tfin) & onehot
        cnt_gt = jnp.sum(jnp.where(gt_m, 1.0, 0.0), axis=1, keepdims=True)
        eqc = jnp.sum(jnp.where(eq_m, 1.0, 0.0), axis=1, keepdims=True)
        need = kq - cnt_gt

        gi = lax.broadcasted_iota(jnp.int32, (G, G), 0)
        gj = lax.broadcasted_iota(jnp.int32, (G, G), 1)
        tri_g = jnp.where(gj < gi, 1.0, 0.0)
        base = jnp.dot(tri_g, eqc, preferred_element_type=_f32)
        lim = base + need

        eq_row = jnp.sum(jnp.where(eq_m, 1.0, 0.0), axis=0, keepdims=True)
        eq2d = eq_row.reshape(NPR, 128)
        ci = lax.broadcasted_iota(jnp.int32, (128, 128), 0)
        cj = lax.broadcasted_iota(jnp.int32, (128, 128), 1)
        tri_incl = jnp.where(ci <= cj, 1.0, 0.0)
        cum_in_row = jnp.dot(eq2d, tri_incl, preferred_element_type=_f32)
        rowtot = jnp.sum(eq2d, axis=1, keepdims=True)
        ri = lax.broadcasted_iota(jnp.int32, (NPR, NPR), 0)
        rj = lax.broadcasted_iota(jnp.int32, (NPR, NPR), 1)
        tri_r = jnp.where(rj < ri, 1.0, 0.0)
        rowbase = jnp.dot(tri_r, rowtot, preferred_element_type=_f32)
        p_excl = cum_in_row + rowbase - eq2d
        p_row = p_excl.reshape(1, NP)

        sel_m = gt_m | (eq_m & (p_row < lim))
        sel_row = jnp.sum(jnp.where(sel_m, 1.0, 0.0), axis=0, keepdims=True)
        w_row = jnp.tanh(score_row) * sel_row
        w_scr[...] = w_row.reshape(NPR, 128)
        cnt_scr[...] = jnp.broadcast_to(kq, (G, 128))
        S_scr[...] = jnp.zeros((G, H), _f32)
        Q_scr[...] = jnp.zeros((8, H), _f32)

    hb = h_ref[...]
    wcol = jnp.transpose(w_scr[pl.ds(pid * 8, 8), :].reshape(1, BR), (1, 0))
    brow = batch_ref[pl.ds(pid * 8, 8), :].reshape(1, BR)
    g_col = lax.broadcasted_iota(jnp.int32, (G, 1), 0)
    onehot_t = jnp.where(brow == g_col, 1.0, 0.0)
    hw = hb * wcol
    S_scr[...] += jnp.dot(onehot_t, hw, preferred_element_type=_f32)
    Q_scr[0:1, :] += jnp.sum(hw * hw, axis=0, keepdims=True)

    @pl.when(pid == NB - 1)
    def _finalize():
        S = S_scr[...]
        cnt = cnt_scr[:, 0:1]
        m_tot = jnp.sum(cnt)
        mean = jnp.sum(S, axis=0, keepdims=True) / m_tot
        var = Q_scr[0:1, :] / m_tot - mean * mean
        gm = gamma_ref[...][0:1, :] * lax.rsqrt(var + 1e-5)
        cntc = jnp.maximum(cnt, 1.0)
        pooled = (S / cntc - mean) * gm + beta_ref[...][0:1, :]
        pooled = jnp.where(cnt > 0.0, pooled, 0.0)
        logits = jnp.dot(pooled, Wfc_ref[...], preferred_element_type=_f32)
        logits = logits + bfc_ref[...][0:1, :]
        lmask = lax.broadcasted_iota(jnp.int32, (1, 128), 1) < C
        lg = jnp.where(lmask, logits, jnp.float32(-1e30))
        mx = jnp.max(lg, axis=1, keepdims=True)
        z = lg - mx
        ez = jnp.where(lmask, jnp.exp(z), 0.0)
        lse = jnp.log(jnp.sum(ez, axis=1, keepdims=True))
        out_ref[...] = z - lse


def _tc_final(hp, s1_2d, sg0, sg1, batchp, bs2d, gm2d, bt2d, Wfc, bfc2d):
    blk = lambda i: (i, 0)
    zero = lambda i: (0, 0)
    return pl.pallas_call(
        _tc_final_body,
        grid=(NB,),
        in_specs=[
            pl.BlockSpec((BR, H), blk),
            pl.BlockSpec((NPR, 128), zero),
            pl.BlockSpec((NPR, 128), zero),
            pl.BlockSpec((NPR, 128), zero),
            pl.BlockSpec((NPR, 128), zero),
            pl.BlockSpec((NPR, 128), zero),
            pl.BlockSpec((8, H), zero),
            pl.BlockSpec((8, H), zero),
            pl.BlockSpec((H, 128), zero),
            pl.BlockSpec((8, 128), zero),
        ],
        out_specs=pl.BlockSpec((G, 128), zero),
        out_shape=jax.ShapeDtypeStruct((G, 128), _f32),
        scratch_shapes=[
            pltpu.VMEM((NPR, 128), _f32),
            pltpu.VMEM((G, 128), _f32),
            pltpu.VMEM((G, H), _f32),
            pltpu.VMEM((8, H), _f32),
        ],
    )(hp, s1_2d, sg0, sg1, batchp, bs2d, gm2d, bt2d, Wfc, bfc2d)


def _to2d(v):
    return v.reshape(NPR, 128)


def kernel(x, edge_index, batch, W_enc, b_enc, W_root, W_rel, b_score,
           gamma, beta, W_fc, b_fc):
    src = edge_index[0]
    dst = edge_index[1]
    npad = EPAD - E
    padidx = (jnp.arange(npad, dtype=jnp.int32) % 48) + N
    srcp = jnp.concatenate([src, padidx]).reshape(NW, CH, CW)
    dstp = jnp.concatenate([dst, padidx]).reshape(NW, CH, CW)

    xp = jnp.pad(x, ((0, NP - N), (0, 0)))
    batchp = jnp.pad(batch, (0, NP - N), constant_values=G).reshape(NPR, 128)

    deg_parts = _sc_deg(dstp)
    d0 = _to2d(deg_parts[:NSC])
    d1 = _to2d(deg_parts[NSC:])

    xs, dinv2d = _tc_scale(xp, d0, d1)

    agg_parts = _sc_agg(srcp, dstp, xs)
    a0 = agg_parts[:NSC]
    a1 = agg_parts[NSC:]

    be2d = jnp.broadcast_to(b_enc.reshape(1, H), (8, H))
    Wrr = jnp.pad(jnp.concatenate([W_root, W_rel], axis=1), ((0, 0), (0, 126)))
    hp, s12 = _tc_encode(a0, a1, xp, dinv2d, W_enc, be2d, Wrr)

    s2p = jnp.pad(s12[:N, 1], (0, NSC - N))
    sagg_parts = _sc_sagg(srcp, dstp, s2p)
    sg0 = _to2d(sagg_parts[:NSC])
    sg1 = _to2d(sagg_parts[NSC:])

    s1_2d = jnp.pad(s12[:N, 0], (0, NP - N)).reshape(NPR, 128)
    bs2d = jnp.broadcast_to(b_score.reshape(1, 1), (NPR, 128))
    gm2d = jnp.broadcast_to(gamma.reshape(1, H), (8, H))
    bt2d = jnp.broadcast_to(beta.reshape(1, H), (8, H))
    Wfc = jnp.pad(W_fc, ((0, 0), (0, 128 - C)))
    bfc2d = jnp.broadcast_to(jnp.pad(b_fc, (0, 128 - C)).reshape(1, 128),
                             (8, 128))

    out2d = _tc_final(hp, s1_2d, sg0, sg1, batchp, bs2d, gm2d, bt2d,
                      Wfc, bfc2d)
    return out2d[:, :C]
